# baseline (speedup 1.0000x reference)
import functools

import jax
import jax.numpy as jnp
from jax import lax
from jax.experimental import pallas as pl
from jax.experimental.pallas import tpu as pltpu
from jax.experimental.pallas import tpu_sc as plsc

VOCAB = 1000000
EMB = 64
PAIR = 2 * EMB
B = 16384 * 20
NC, NS = 2, 16
NW = NC * NS
B_PER_W = B // NW
CHUNK = 160
N_CHUNKS = B_PER_W // CHUNK
NBUF = 4

_mesh = plsc.VectorSubcoreMesh(
    core_axis_name="c", subcore_axis_name="s", num_cores=NC, num_subcores=NS
)


@functools.partial(
    pl.kernel,
    out_type=jax.ShapeDtypeStruct((B, PAIR), jnp.float32),
    mesh=_mesh,
    scratch_types=[
        pltpu.VMEM((B_PER_W,), jnp.int32),
        [pltpu.VMEM((CHUNK, PAIR), jnp.float32) for _ in range(NBUF)],
        [pltpu.SemaphoreType.DMA for _ in range(NBUF)],
        [pltpu.SemaphoreType.DMA for _ in range(NBUF)],
    ],
)
def _gather(table_hbm, idx_hbm, out_hbm, idx_v, rows, gsem, ssem):
    wid = lax.axis_index("s") * NC + lax.axis_index("c")
    base = wid * B_PER_W
    pltpu.sync_copy(idx_hbm.at[pl.ds(base, B_PER_W)], idx_v)

    def start_gather(c, b):
        return pltpu.async_copy(
            table_hbm.at[idx_v.at[pl.ds(c * CHUNK, CHUNK)]], rows[b], gsem[b]
        )

    def start_store(c, b):
        return pltpu.async_copy(
            rows[b], out_hbm.at[pl.ds(base + c * CHUNK, CHUNK)], ssem[b]
        )

    gaths = [None] * NBUF
    stores = [None] * NBUF
    gaths[0] = start_gather(0, 0)
    for c in range(N_CHUNKS):
        b = c % NBUF
        nxt = c + 1
        if nxt < N_CHUNKS:
            nb = nxt % NBUF
            if stores[nb] is not None:
                stores[nb].wait()
                stores[nb] = None
            gaths[nb] = start_gather(nxt, nb)
        gaths[b].wait()
        stores[b] = start_store(c, b)
    for b in range(NBUF):
        if stores[b] is not None:
            stores[b].wait()


def kernel(token_ids, weight):
    wp = weight.reshape(VOCAB // 2, PAIR)
    flat = token_ids.reshape(-1)
    pairs = _gather(wp, (flat >> 1).astype(jnp.int32))
    h = (flat & 1) == 1
    out = jnp.where(h[:, None], pairs[:, EMB:], pairs[:, :EMB])
    return out.reshape(token_ids.shape + (EMB,))


# speedup vs baseline: 1.3472x; 1.3472x over previous
import functools

import jax
import jax.numpy as jnp
from jax import lax
from jax.experimental import pallas as pl
from jax.experimental.pallas import tpu as pltpu
from jax.experimental.pallas import tpu_sc as plsc

VOCAB = 1000000
EMB = 64
B = 16384 * 20
NC, NS = 2, 16
NW = NC * NS
B_PER_W = B // NW
CHUNK = 320
N_CHUNKS = B_PER_W // CHUNK
NBUF = 4

_mesh = plsc.VectorSubcoreMesh(
    core_axis_name="c", subcore_axis_name="s", num_cores=NC, num_subcores=NS
)


@functools.partial(
    pl.kernel,
    out_type=jax.ShapeDtypeStruct((B, EMB), jnp.float32),
    mesh=_mesh,
    scratch_types=[
        pltpu.VMEM((B_PER_W,), jnp.int32),
        [pltpu.VMEM((CHUNK, EMB), jnp.float32) for _ in range(NBUF)],
        [pltpu.SemaphoreType.DMA for _ in range(NBUF)],
        [pltpu.SemaphoreType.DMA for _ in range(NBUF)],
    ],
    compiler_params=pltpu.CompilerParams(
        use_tc_tiling_on_sc=False, skip_device_barrier=True
    ),
)
def _gather(table_hbm, idx_hbm, out_hbm, idx_v, rows, gsem, ssem):
    wid = lax.axis_index("s") * NC + lax.axis_index("c")
    base = wid * B_PER_W
    pltpu.sync_copy(idx_hbm.at[pl.ds(base, B_PER_W)], idx_v)

    def start_gather(c, b):
        return pltpu.async_copy(
            table_hbm.at[idx_v.at[pl.ds(c * CHUNK, CHUNK)]], rows[b], gsem[b]
        )

    def start_store(c, b):
        return pltpu.async_copy(
            rows[b], out_hbm.at[pl.ds(base + c * CHUNK, CHUNK)], ssem[b]
        )

    gaths = [None] * NBUF
    stores = [None] * NBUF
    gaths[0] = start_gather(0, 0)
    for c in range(N_CHUNKS):
        b = c % NBUF
        nxt = c + 1
        if nxt < N_CHUNKS:
            nb = nxt % NBUF
            if stores[nb] is not None:
                stores[nb].wait()
                stores[nb] = None
            gaths[nb] = start_gather(nxt, nb)
        gaths[b].wait()
        stores[b] = start_store(c, b)
    for b in range(NBUF):
        if stores[b] is not None:
            stores[b].wait()


def kernel(token_ids, weight):
    out = _gather(weight, token_ids.reshape(-1))
    return out.reshape(token_ids.shape + (EMB,))
